# Initial kernel scaffold; baseline (speedup 1.0000x reference)
import jax
import jax.numpy as jnp
from jax.experimental import pallas as pl


def kernel(x, logits):
    raise NotImplementedError("write your pallas kernel here")


# speedup vs baseline: 1179.0156x; 1179.0156x over previous
import functools
import math

import jax
import jax.numpy as jnp
from jax import lax
from jax.experimental import pallas as pl
from jax.experimental.pallas import tpu as pltpu
from jax.experimental.pallas import tpu_sc as plsc

N_BINS = 8192
LOG_N_BINS = math.log(N_BINS)

NC = 2
NS = 16
NW = NC * NS
LANES = 16

CHUNK = 16384


def _table_body(logits_ref, out_ref):
    l = logits_ref[...]
    m = jnp.max(l)
    lse = jnp.log(jnp.sum(jnp.exp(l - m))) + m
    out_ref[...] = l - lse + LOG_N_BINS


def _build_table(logits):
    l2d = logits.reshape(64, 128)
    out = pl.pallas_call(
        _table_body,
        out_shape=jax.ShapeDtypeStruct((64, 128), jnp.float32),
    )(l2d)
    return out.reshape(N_BINS)


def _sc_body(n_samples, x_hbm, tab_hbm, out_hbm, tab_v, x_v, out_v):
    per_w = n_samples // NW
    c = lax.axis_index("c")
    s = lax.axis_index("s")
    wid = s * NC + c
    base = wid * per_w

    pltpu.sync_copy(tab_hbm, tab_v)

    def chunk_body(j, _):
        off = base + j * CHUNK
        pltpu.sync_copy(x_hbm.at[pl.ds(off, CHUNK)], x_v)

        @plsc.parallel_loop(0, CHUNK, step=LANES, unroll=8)
        def _(i):
            xv = x_v[pl.ds(i, LANES)]
            idx = (xv * float(N_BINS)).astype(jnp.int32)
            idx = jnp.minimum(jnp.maximum(idx, 0), N_BINS - 1)
            out_v[pl.ds(i, LANES)] = plsc.load_gather(tab_v, [idx])

        pltpu.sync_copy(out_v, out_hbm.at[pl.ds(off, CHUNK)])
        return 0

    lax.fori_loop(0, per_w // CHUNK, chunk_body, 0)


def kernel(x, logits):
    n = x.shape[0]
    table = _build_table(logits)
    mesh = plsc.VectorSubcoreMesh(core_axis_name="c", subcore_axis_name="s")
    sc = pl.kernel(
        functools.partial(_sc_body, n),
        out_type=jax.ShapeDtypeStruct((n,), jnp.float32),
        mesh=mesh,
        compiler_params=pltpu.CompilerParams(needs_layout_passes=False),
        scratch_types=[
            pltpu.VMEM((N_BINS,), jnp.float32),
            pltpu.VMEM((CHUNK,), jnp.float32),
            pltpu.VMEM((CHUNK,), jnp.float32),
        ],
    )
    return sc(x, table)


# speedup vs baseline: 1436.9031x; 1.2187x over previous
import functools
import math

import jax
import jax.numpy as jnp
from jax import lax
from jax.experimental import pallas as pl
from jax.experimental.pallas import tpu as pltpu
from jax.experimental.pallas import tpu_sc as plsc

N_BINS = 8192
LOG_N_BINS = math.log(N_BINS)

NC = 2
NS = 16
NW = NC * NS
LANES = 16

CHUNK = 16384
NBUF = 2


def _table_body(logits_ref, out_ref):
    l = logits_ref[...]
    m = jnp.max(l)
    lse = jnp.log(jnp.sum(jnp.exp(l - m))) + m
    out_ref[...] = l - lse + LOG_N_BINS


def _build_table(logits):
    l2d = logits.reshape(64, 128)
    out = pl.pallas_call(
        _table_body,
        out_shape=jax.ShapeDtypeStruct((64, 128), jnp.float32),
    )(l2d)
    return out.reshape(N_BINS)


def _sc_body(n_samples, x_hbm, tab_hbm, out_hbm, tab_v, x_v, out_v,
             sems_in, sems_out):
    per_w = n_samples // NW
    n_chunks = per_w // CHUNK
    c = lax.axis_index("c")
    s = lax.axis_index("s")
    wid = s * NC + c
    base = wid * per_w

    pltpu.sync_copy(tab_hbm, tab_v)

    for b in range(NBUF):
        pltpu.async_copy(
            x_hbm.at[pl.ds(base + b * CHUNK, CHUNK)], x_v.at[b], sems_in.at[b])

    def ring_body(j, _):
        for b in range(NBUF):
            ch = j * NBUF + b
            off = base + ch * CHUNK
            pltpu.make_async_copy(
                x_hbm.at[pl.ds(base, CHUNK)], x_v.at[b], sems_in.at[b]).wait()
            @pl.when(j > 0)
            def _():
                pltpu.make_async_copy(
                    out_v.at[b], out_hbm.at[pl.ds(base, CHUNK)],
                    sems_out.at[b]).wait()

            @plsc.parallel_loop(0, CHUNK, step=LANES, unroll=8)
            def _(i):
                xv = x_v[b, pl.ds(i, LANES)]
                idx = (xv * float(N_BINS)).astype(jnp.int32)
                idx = jnp.minimum(jnp.maximum(idx, 0), N_BINS - 1)
                out_v[b, pl.ds(i, LANES)] = plsc.load_gather(tab_v, [idx])

            pltpu.async_copy(out_v.at[b], out_hbm.at[pl.ds(off, CHUNK)],
                             sems_out.at[b])
            @pl.when(ch + NBUF < n_chunks)
            def _():
                pltpu.async_copy(
                    x_hbm.at[pl.ds(off + NBUF * CHUNK, CHUNK)], x_v.at[b],
                    sems_in.at[b])
        return 0

    lax.fori_loop(0, n_chunks // NBUF, ring_body, 0)

    for b in range(NBUF):
        pltpu.make_async_copy(
            out_v.at[b], out_hbm.at[pl.ds(base, CHUNK)], sems_out.at[b]).wait()


def kernel(x, logits):
    n = x.shape[0]
    table = _build_table(logits)
    mesh = plsc.VectorSubcoreMesh(core_axis_name="c", subcore_axis_name="s")
    sc = pl.kernel(
        functools.partial(_sc_body, n),
        out_type=jax.ShapeDtypeStruct((n,), jnp.float32),
        mesh=mesh,
        compiler_params=pltpu.CompilerParams(needs_layout_passes=False),
        scratch_types=[
            pltpu.VMEM((N_BINS,), jnp.float32),
            pltpu.VMEM((NBUF, CHUNK), jnp.float32),
            pltpu.VMEM((NBUF, CHUNK), jnp.float32),
            pltpu.SemaphoreType.DMA((NBUF,)),
            pltpu.SemaphoreType.DMA((NBUF,)),
        ],
    )
    return sc(x, table)


# speedup vs baseline: 1467.0125x; 1.0210x over previous
import functools
import math

import jax
import jax.numpy as jnp
from jax import lax
from jax.experimental import pallas as pl
from jax.experimental.pallas import tpu as pltpu
from jax.experimental.pallas import tpu_sc as plsc

N_BINS = 8192
LOG_N_BINS = math.log(N_BINS)

NC = 2
NS = 16
NW = NC * NS
LANES = 16

CHUNK = 16384
NBUF = 2


def _table_body(logits_ref, out_ref):
    l = logits_ref[...]
    m = jnp.max(l)
    lse = jnp.log(jnp.sum(jnp.exp(l - m))) + m
    out_ref[...] = l - lse + LOG_N_BINS


def _build_table(logits):
    l2d = logits.reshape(64, 128)
    out = pl.pallas_call(
        _table_body,
        out_shape=jax.ShapeDtypeStruct((64, 128), jnp.float32),
    )(l2d)
    return out.reshape(N_BINS)


def _sc_body(n_samples, x_hbm, tab_hbm, out_hbm, tab_v, x_v, out_v,
             sems_in, sems_out):
    per_w = n_samples // NW
    n_chunks = per_w // CHUNK
    c = lax.axis_index("c")
    s = lax.axis_index("s")
    wid = s * NC + c
    base = wid * per_w

    pltpu.sync_copy(tab_hbm, tab_v)

    for b in range(NBUF):
        pltpu.async_copy(
            x_hbm.at[pl.ds(base + b * CHUNK, CHUNK)], x_v.at[b], sems_in.at[b])

    def ring_body(j, _):
        for b in range(NBUF):
            ch = j * NBUF + b
            off = base + ch * CHUNK
            pltpu.make_async_copy(
                x_hbm.at[pl.ds(base, CHUNK)], x_v.at[b], sems_in.at[b]).wait()
            @pl.when(j > 0)
            def _():
                pltpu.make_async_copy(
                    out_v.at[b], out_hbm.at[pl.ds(base, CHUNK)],
                    sems_out.at[b]).wait()

            @plsc.parallel_loop(0, CHUNK, step=LANES, unroll=16)
            def _(i):
                xv = x_v[b, pl.ds(i, LANES)]
                idx = (xv * float(N_BINS)).astype(jnp.int32)
                idx = jnp.minimum(jnp.maximum(idx, 0), N_BINS - 1)
                out_v[b, pl.ds(i, LANES)] = plsc.load_gather(tab_v, [idx])

            pltpu.async_copy(out_v.at[b], out_hbm.at[pl.ds(off, CHUNK)],
                             sems_out.at[b])
            @pl.when(ch + NBUF < n_chunks)
            def _():
                pltpu.async_copy(
                    x_hbm.at[pl.ds(off + NBUF * CHUNK, CHUNK)], x_v.at[b],
                    sems_in.at[b])
        return 0

    lax.fori_loop(0, n_chunks // NBUF, ring_body, 0)

    for b in range(NBUF):
        pltpu.make_async_copy(
            out_v.at[b], out_hbm.at[pl.ds(base, CHUNK)], sems_out.at[b]).wait()


def kernel(x, logits):
    n = x.shape[0]
    table = _build_table(logits)
    mesh = plsc.VectorSubcoreMesh(core_axis_name="c", subcore_axis_name="s")
    sc = pl.kernel(
        functools.partial(_sc_body, n),
        out_type=jax.ShapeDtypeStruct((n,), jnp.float32),
        mesh=mesh,
        compiler_params=pltpu.CompilerParams(needs_layout_passes=False),
        scratch_types=[
            pltpu.VMEM((N_BINS,), jnp.float32),
            pltpu.VMEM((NBUF, CHUNK), jnp.float32),
            pltpu.VMEM((NBUF, CHUNK), jnp.float32),
            pltpu.SemaphoreType.DMA((NBUF,)),
            pltpu.SemaphoreType.DMA((NBUF,)),
        ],
    )
    return sc(x, table)
